# baseline (device time: 35547 ns/iter reference)
import jax
import jax.numpy as jnp
from jax import lax
from jax.experimental import pallas as pl
from jax.experimental.pallas import tpu as pltpu

B, S, H_SHARD, D = 4, 512, 8, 64
K = H_SHARD * D
N = 1024
S_HALF = S // 2


def kernel(O, Wo):
    O2 = O.reshape(B * S, H_SHARD, D)

    def body(o_ref, wo_ref, out_ref, o_vmem, out_vmem, send_ref, recv_ref,
             load_sems, out_sems, send_sems, recv_sems):
        my_x = lax.axis_index("x")
        my_y = lax.axis_index("y")
        my_z = lax.axis_index("z")
        partner = 1 - my_x

        barrier_sem = pltpu.get_barrier_semaphore()
        pl.semaphore_signal(
            barrier_sem, inc=1,
            device_id=(partner, my_y, my_z),
            device_id_type=pl.DeviceIdType.MESH,
        )
        pl.semaphore_wait(barrier_sem, 1)

        partner_s = partner * S_HALF
        my_s = my_x * S_HALF

        loads = []
        for c in range(2 * B):
            b = c % B
            s0 = partner_s if c < B else my_s
            cp = pltpu.make_async_copy(
                o_ref.at[pl.ds(b * S + s0, S_HALF)], o_vmem.at[c],
                load_sems.at[c],
            )
            cp.start()
            loads.append(cp)

        w = wo_ref[...].astype(jnp.bfloat16)

        rdmas = []
        for b in range(B):
            loads[b].wait()
            ob = o_vmem[b].reshape(S_HALF, K).astype(jnp.bfloat16)
            send_ref[b, :, :] = jnp.dot(
                ob, w, preferred_element_type=jnp.float32
            ).astype(jnp.bfloat16)
            rdma = pltpu.make_async_remote_copy(
                src_ref=send_ref.at[b],
                dst_ref=recv_ref.at[b],
                send_sem=send_sems.at[b],
                recv_sem=recv_sems.at[b],
                device_id=(partner, my_y, my_z),
                device_id_type=pl.DeviceIdType.MESH,
            )
            rdma.start()
            rdmas.append(rdma)

        outs = []
        for b in range(B):
            loads[B + b].wait()
            ob = o_vmem[B + b].reshape(S_HALF, K).astype(jnp.bfloat16)
            mine = jnp.dot(ob, w, preferred_element_type=jnp.float32)
            rdmas[b].wait_recv()
            out_vmem[b, :, :] = mine + recv_ref[b, :, :].astype(jnp.float32)
            cp = pltpu.make_async_copy(
                out_vmem.at[b], out_ref.at[b], out_sems.at[b]
            )
            cp.start()
            outs.append(cp)

        for b in range(B):
            outs[b].wait()
            rdmas[b].wait_send()

    return pl.pallas_call(
        body,
        out_shape=jax.ShapeDtypeStruct((B, S_HALF, N), jnp.float32),
        in_specs=[
            pl.BlockSpec(memory_space=pl.MemorySpace.ANY),
            pl.BlockSpec(memory_space=pltpu.MemorySpace.VMEM),
        ],
        out_specs=pl.BlockSpec(memory_space=pl.MemorySpace.ANY),
        scratch_shapes=[
            pltpu.VMEM((2 * B, S_HALF, H_SHARD, D), jnp.float32),
            pltpu.VMEM((B, S_HALF, N), jnp.float32),
            pltpu.VMEM((B, S_HALF, N), jnp.bfloat16),
            pltpu.VMEM((B, S_HALF, N), jnp.bfloat16),
            pltpu.SemaphoreType.DMA((2 * B,)),
            pltpu.SemaphoreType.DMA((B,)),
            pltpu.SemaphoreType.DMA((B,)),
            pltpu.SemaphoreType.DMA((B,)),
        ],
        compiler_params=pltpu.CompilerParams(collective_id=0),
    )(O2, Wo)


# device time: 34621 ns/iter; 1.0267x vs baseline; 1.0267x over previous
import jax
import jax.numpy as jnp
from jax import lax
from jax.experimental import pallas as pl
from jax.experimental.pallas import tpu as pltpu

B, S, H_SHARD, D = 4, 512, 8, 64
K = H_SHARD * D
N = 1024
S_HALF = S // 2
R = 128
C = (B * S_HALF) // R


def kernel(O, Wo):
    O2 = O.reshape(B * S, H_SHARD, D)

    def body(o_ref, wo_ref, out_ref, send_ref, recv_ref, send_sems, recv_sems):
        my_x = lax.axis_index("x")
        my_y = lax.axis_index("y")
        my_z = lax.axis_index("z")
        partner = 1 - my_x

        barrier_sem = pltpu.get_barrier_semaphore()
        pl.semaphore_signal(
            barrier_sem, inc=1,
            device_id=(partner, my_y, my_z),
            device_id_type=pl.DeviceIdType.MESH,
        )
        pl.semaphore_wait(barrier_sem, 1)

        w = wo_ref[...].astype(jnp.bfloat16)
        partner_s = partner * S_HALF
        my_s = my_x * S_HALF

        rdmas = []
        for c in range(C):
            b, sub = c // 2, c % 2
            ob = (
                o_ref[pl.ds(b * S + partner_s + sub * R, R), :, :]
                .reshape(R, K)
                .astype(jnp.bfloat16)
            )
            send_ref[c, :, :] = jnp.dot(
                ob, w, preferred_element_type=jnp.float32
            ).astype(jnp.bfloat16)
            rdma = pltpu.make_async_remote_copy(
                src_ref=send_ref.at[c],
                dst_ref=recv_ref.at[b, pl.ds(sub * R, R)],
                send_sem=send_sems.at[c],
                recv_sem=recv_sems.at[c],
                device_id=(partner, my_y, my_z),
                device_id_type=pl.DeviceIdType.MESH,
            )
            rdma.start()
            rdmas.append(rdma)

        for b in range(B):
            ob = (
                o_ref[pl.ds(b * S + my_s, S_HALF), :, :]
                .reshape(S_HALF, K)
                .astype(jnp.bfloat16)
            )
            mine = jnp.dot(ob, w, preferred_element_type=jnp.float32)
            rdmas[2 * b].wait_recv()
            rdmas[2 * b + 1].wait_recv()
            out_ref[b, :, :] = mine + recv_ref[b, :, :].astype(jnp.float32)

        for c in range(C):
            rdmas[c].wait_send()

    return pl.pallas_call(
        body,
        out_shape=jax.ShapeDtypeStruct((B, S_HALF, N), jnp.float32),
        in_specs=[
            pl.BlockSpec(memory_space=pltpu.MemorySpace.VMEM),
            pl.BlockSpec(memory_space=pltpu.MemorySpace.VMEM),
        ],
        out_specs=pl.BlockSpec(memory_space=pltpu.MemorySpace.VMEM),
        scratch_shapes=[
            pltpu.VMEM((C, R, N), jnp.bfloat16),
            pltpu.VMEM((B, S_HALF, N), jnp.bfloat16),
            pltpu.SemaphoreType.DMA((C,)),
            pltpu.SemaphoreType.DMA((C,)),
        ],
        compiler_params=pltpu.CompilerParams(collective_id=0),
    )(O2, Wo)


# device time: 28262 ns/iter; 1.2578x vs baseline; 1.2250x over previous
import jax
import jax.numpy as jnp
from jax import lax
from jax.experimental import pallas as pl
from jax.experimental.pallas import tpu as pltpu

B, S, H_SHARD, D = 4, 512, 8, 64
K = H_SHARD * D
N = 1024
S_HALF = S // 2
R = 128
NB = B // 2
XC = NB * 2
ZC = NB * 2


def kernel(O, Wo):
    O2 = O.reshape(B * S, H_SHARD, D)

    def body(o_ref, wo_ref, out_ref, send_x, recv_x, send_z, recv_z,
             sx_sems, rx_sems, sz_sems, rz_sems):
        my_x = lax.axis_index("x")
        my_y = lax.axis_index("y")
        my_z = lax.axis_index("z")
        px = 1 - my_x
        pz = 1 - my_z

        barrier_sem = pltpu.get_barrier_semaphore()
        pl.semaphore_signal(
            barrier_sem, inc=1,
            device_id=(px, my_y, my_z), device_id_type=pl.DeviceIdType.MESH,
        )
        pl.semaphore_signal(
            barrier_sem, inc=1,
            device_id=(my_x, my_y, pz), device_id_type=pl.DeviceIdType.MESH,
        )
        pl.semaphore_wait(barrier_sem, 2)

        w = wo_ref[...].astype(jnp.bfloat16)
        partner_s = px * S_HALF
        my_s = my_x * S_HALF
        base_b = NB * my_z
        other_b = NB * pz

        x_rdmas = []
        for c in range(XC):
            i, sub = c // 2, c % 2
            row0 = (base_b + i) * S + partner_s + sub * R
            ob = (
                o_ref[pl.ds(row0, R), :, :]
                .reshape(R, K)
                .astype(jnp.bfloat16)
            )
            send_x[c, :, :] = jnp.dot(
                ob, w, preferred_element_type=jnp.float32
            ).astype(jnp.bfloat16)
            rdma = pltpu.make_async_remote_copy(
                src_ref=send_x.at[c],
                dst_ref=recv_x.at[c],
                send_sem=sx_sems.at[c],
                recv_sem=rx_sems.at[c],
                device_id=(px, my_y, my_z),
                device_id_type=pl.DeviceIdType.MESH,
            )
            rdma.start()
            x_rdmas.append(rdma)

        z_rdmas = []
        for i in range(NB):
            b = base_b + i
            ob = (
                o_ref[pl.ds(b * S + my_s, S_HALF), :, :]
                .reshape(S_HALF, K)
                .astype(jnp.bfloat16)
            )
            mine = jnp.dot(ob, w, preferred_element_type=jnp.float32)
            for sub in range(2):
                c = 2 * i + sub
                x_rdmas[c].wait_recv()
                fin = (
                    mine[sub * R:(sub + 1) * R, :]
                    + recv_x[c, :, :].astype(jnp.float32)
                )
                out_ref[pl.ds(b * S_HALF + sub * R, R), :] = fin
                send_z[c, :, :] = fin.astype(jnp.bfloat16)
                rdma = pltpu.make_async_remote_copy(
                    src_ref=send_z.at[c],
                    dst_ref=recv_z.at[c],
                    send_sem=sz_sems.at[c],
                    recv_sem=rz_sems.at[c],
                    device_id=(my_x, my_y, pz),
                    device_id_type=pl.DeviceIdType.MESH,
                )
                rdma.start()
                z_rdmas.append(rdma)

        for c in range(ZC):
            i, sub = c // 2, c % 2
            b = other_b + i
            z_rdmas[c].wait_recv()
            out_ref[pl.ds(b * S_HALF + sub * R, R), :] = (
                recv_z[c, :, :].astype(jnp.float32)
            )

        for c in range(XC):
            x_rdmas[c].wait_send()
        for c in range(ZC):
            z_rdmas[c].wait_send()

    out2 = pl.pallas_call(
        body,
        out_shape=jax.ShapeDtypeStruct((B * S_HALF, N), jnp.float32),
        in_specs=[
            pl.BlockSpec(memory_space=pltpu.MemorySpace.VMEM),
            pl.BlockSpec(memory_space=pltpu.MemorySpace.VMEM),
        ],
        out_specs=pl.BlockSpec(memory_space=pltpu.MemorySpace.VMEM),
        scratch_shapes=[
            pltpu.VMEM((XC, R, N), jnp.bfloat16),
            pltpu.VMEM((XC, R, N), jnp.bfloat16),
            pltpu.VMEM((ZC, R, N), jnp.bfloat16),
            pltpu.VMEM((ZC, R, N), jnp.bfloat16),
            pltpu.SemaphoreType.DMA((XC,)),
            pltpu.SemaphoreType.DMA((XC,)),
            pltpu.SemaphoreType.DMA((ZC,)),
            pltpu.SemaphoreType.DMA((ZC,)),
        ],
        compiler_params=pltpu.CompilerParams(collective_id=0),
    )(O2, Wo)
    return out2.reshape(B, S_HALF, N)


# device time: 27639 ns/iter; 1.2861x vs baseline; 1.0225x over previous
import jax
import jax.numpy as jnp
from jax import lax
from jax.experimental import pallas as pl
from jax.experimental.pallas import tpu as pltpu

B, S, H_SHARD, D = 4, 512, 8, 64
K = H_SHARD * D
N = 1024
S_HALF = S // 2
R = 128
RZ = 64
NB = B // 2
XC = NB * 2
ZC = NB * 4


def kernel(O, Wo):
    O2 = O.reshape(B * S, H_SHARD, D)

    def body(o_ref, wo_ref, out_ref, send_x, recv_x, send_z, recv_z,
             sx_sems, rx_sems, sz_sems, rz_sems):
        my_x = lax.axis_index("x")
        my_y = lax.axis_index("y")
        my_z = lax.axis_index("z")
        px = 1 - my_x
        pz = 1 - my_z

        barrier_sem = pltpu.get_barrier_semaphore()
        pl.semaphore_signal(
            barrier_sem, inc=1,
            device_id=(px, my_y, my_z), device_id_type=pl.DeviceIdType.MESH,
        )
        pl.semaphore_signal(
            barrier_sem, inc=1,
            device_id=(my_x, my_y, pz), device_id_type=pl.DeviceIdType.MESH,
        )
        pl.semaphore_wait(barrier_sem, 2)

        w = wo_ref[...].astype(jnp.bfloat16)
        partner_s = px * S_HALF
        my_s = my_x * S_HALF
        base_b = NB * my_z
        other_b = NB * pz

        x_rdmas = []
        for c in range(XC):
            i, sub = c // 2, c % 2
            row0 = (base_b + i) * S + partner_s + sub * R
            ob = (
                o_ref[pl.ds(row0, R), :, :]
                .reshape(R, K)
                .astype(jnp.bfloat16)
            )
            send_x[c, :, :] = jnp.dot(
                ob, w, preferred_element_type=jnp.float32
            ).astype(jnp.bfloat16)
            rdma = pltpu.make_async_remote_copy(
                src_ref=send_x.at[c],
                dst_ref=recv_x.at[c],
                send_sem=sx_sems.at[c],
                recv_sem=rx_sems.at[c],
                device_id=(px, my_y, my_z),
                device_id_type=pl.DeviceIdType.MESH,
            )
            rdma.start()
            x_rdmas.append(rdma)

        z_rdmas = []
        for i in range(NB):
            b = base_b + i
            ob = (
                o_ref[pl.ds(b * S + my_s, S_HALF), :, :]
                .reshape(S_HALF, K)
                .astype(jnp.bfloat16)
            )
            mine = jnp.dot(ob, w, preferred_element_type=jnp.float32)
            for sub in range(2):
                c = 2 * i + sub
                x_rdmas[c].wait_recv()
                fin = (
                    mine[sub * R:(sub + 1) * R, :]
                    + recv_x[c, :, :].astype(jnp.float32)
                ).astype(jnp.bfloat16)
                out_ref[pl.ds(b * S_HALF + sub * R, R), :] = fin
                for h in range(2):
                    zc = 2 * c + h
                    send_z[zc, :, :] = fin[h * RZ:(h + 1) * RZ, :]
                    rdma = pltpu.make_async_remote_copy(
                        src_ref=send_z.at[zc],
                        dst_ref=recv_z.at[zc],
                        send_sem=sz_sems.at[zc],
                        recv_sem=rz_sems.at[zc],
                        device_id=(my_x, my_y, pz),
                        device_id_type=pl.DeviceIdType.MESH,
                    )
                    rdma.start()
                    z_rdmas.append(rdma)

        for zc in range(ZC):
            i = zc // 4
            b = other_b + i
            off = (zc % 4) * RZ
            z_rdmas[zc].wait_recv()
            out_ref[pl.ds(b * S_HALF + off, RZ), :] = recv_z[zc, :, :]

        for c in range(XC):
            x_rdmas[c].wait_send()
        for zc in range(ZC):
            z_rdmas[zc].wait_send()

    out2 = pl.pallas_call(
        body,
        out_shape=jax.ShapeDtypeStruct((B * S_HALF, N), jnp.bfloat16),
        in_specs=[
            pl.BlockSpec(memory_space=pltpu.MemorySpace.VMEM),
            pl.BlockSpec(memory_space=pltpu.MemorySpace.VMEM),
        ],
        out_specs=pl.BlockSpec(memory_space=pltpu.MemorySpace.VMEM),
        scratch_shapes=[
            pltpu.VMEM((XC, R, N), jnp.bfloat16),
            pltpu.VMEM((XC, R, N), jnp.bfloat16),
            pltpu.VMEM((ZC, RZ, N), jnp.bfloat16),
            pltpu.VMEM((ZC, RZ, N), jnp.bfloat16),
            pltpu.SemaphoreType.DMA((XC,)),
            pltpu.SemaphoreType.DMA((XC,)),
            pltpu.SemaphoreType.DMA((ZC,)),
            pltpu.SemaphoreType.DMA((ZC,)),
        ],
        compiler_params=pltpu.CompilerParams(collective_id=0),
    )(O2, Wo)
    return out2.reshape(B, S_HALF, N)
